# Initial kernel scaffold; baseline (speedup 1.0000x reference)
import jax
import jax.numpy as jnp
from jax.experimental import pallas as pl


def kernel(x, edge_index, W1, att_src1, att_dst1, b1, W2, att_src2, att_dst2, b2):
    raise NotImplementedError("write your pallas kernel here")


# speedup vs baseline: 18.3423x; 18.3423x over previous
import functools

import jax
import jax.numpy as jnp
from jax import lax
from jax.experimental import pallas as pl
from jax.experimental.pallas import tpu as pltpu
from jax.experimental.pallas import tpu_sc as plsc

NC, NS = 2, 16
NW = NC * NS
BLK = 128
NEG_INIT = -1e30

_SC_PARAMS = pltpu.CompilerParams(
    needs_layout_passes=False, use_tc_tiling_on_sc=False)

_MESH = functools.partial(
    plsc.VectorSubcoreMesh,
    core_axis_name="c", subcore_axis_name="s",
    num_cores=NC, num_subcores=NS,
)


def _f32(shape):
    return jax.ShapeDtypeStruct(shape, jnp.float32)


def _worker_id():
    return lax.axis_index("s") * NC + lax.axis_index("c")


def _iota16():
    return lax.iota(jnp.int32, 16)


def _gmax_vecs(tmax_ref, heads):
    out = []
    for h in range(heads):
        m = tmax_ref[0, h, :]
        for t in range(1, NW):
            m = jnp.maximum(m, tmax_ref[t, h, :])
        out.append(jnp.full((16,), jnp.max(m), jnp.float32))
    return out


def _gmax_vec1(tmax_ref):
    m = tmax_ref[0, :]
    for t in range(1, NW):
        m = jnp.maximum(m, tmax_ref[t, :])
    return jnp.full((16,), jnp.max(m), jnp.float32)


def _sc_a1(nblk_tile, np_rows):
    def body(src_h, dst_h, asad_h, alpha_h, tmax_h,
             idx_s, idx_d, rows_s, rows_d, abuf, mbuf):
        wid = _worker_id()
        iota = _iota16()
        for h in range(8):
            mbuf[h, :] = jnp.full((16,), NEG_INIT, jnp.float32)

        def blk(b, _):
            tb = wid * nblk_tile + b
            base = tb * BLK
            pltpu.sync_copy(src_h.at[pl.ds(base, BLK)], idx_s)
            pltpu.sync_copy(dst_h.at[pl.ds(base, BLK)], idx_d)
            pltpu.sync_copy(asad_h.at[idx_s], rows_s)
            pltpu.sync_copy(asad_h.at[idx_d], rows_d)
            for h in range(8):
                hh_s = jnp.full((16,), h, jnp.int32)
                hh_d = jnp.full((16,), h + 8, jnp.int32)
                mv = mbuf[h, :]
                for g in range(8):
                    ii = g * 16 + iota
                    sv = plsc.load_gather(rows_s, [ii, hh_s])
                    dv = plsc.load_gather(rows_d, [ii, hh_d])
                    v = sv + dv
                    a = jnp.maximum(v, 0.2 * v)
                    mv = jnp.maximum(mv, a)
                    abuf[h, pl.ds(g * 16, 16)] = a
                mbuf[h, :] = mv
            pltpu.sync_copy(abuf, alpha_h.at[tb])
            return 0

        lax.fori_loop(0, nblk_tile, blk, 0)
        pltpu.sync_copy(mbuf, tmax_h.at[wid])

    nblk = nblk_tile * NW
    return pl.kernel(
        body,
        out_type=[_f32((nblk, 8, BLK)), _f32((NW, 8, 16))],
        mesh=_MESH(),
        compiler_params=_SC_PARAMS,
        scratch_types=[
            pltpu.VMEM((BLK,), jnp.int32),
            pltpu.VMEM((BLK,), jnp.int32),
            pltpu.VMEM((BLK, 16), jnp.float32),
            pltpu.VMEM((BLK, 16), jnp.float32),
            pltpu.VMEM((8, BLK), jnp.float32),
            pltpu.VMEM((8, 16), jnp.float32),
        ],
    )


def _sc_b1(nblk_tile, np_rows):
    def body(dst_h, alpha_h, tmax_h, zeros8_h, dparts_h,
             tmax_v, abuf, exbuf, idx_d, shared_d):
        cid = lax.axis_index("c")
        sid = lax.axis_index("s")
        wid = _worker_id()
        iota = _iota16()
        pltpu.sync_copy(tmax_h, tmax_v)
        gvecs = _gmax_vecs(tmax_v, 8)

        @pl.when(sid == 0)
        def _():
            pltpu.sync_copy(zeros8_h, shared_d)
        plsc.subcore_barrier()

        def blk(b, _):
            tb = wid * nblk_tile + b
            base = tb * BLK
            pltpu.sync_copy(dst_h.at[pl.ds(base, BLK)], idx_d)
            pltpu.sync_copy(alpha_h.at[tb], abuf)
            for h in range(8):
                hh = jnp.full((16,), h, jnp.int32)
                for g in range(8):
                    a = abuf[h, pl.ds(g * 16, 16)]
                    ex = jnp.exp(a - gvecs[h])
                    plsc.store_scatter(exbuf, [g * 16 + iota, hh], ex)
            pltpu.sync_copy(exbuf, shared_d.at[idx_d], add=True)
            return 0

        lax.fori_loop(0, nblk_tile, blk, 0)
        plsc.subcore_barrier()

        @pl.when(sid == 0)
        def _():
            pltpu.sync_copy(shared_d, dparts_h.at[cid])

    return pl.kernel(
        body,
        out_type=[_f32((NC, np_rows, 8))],
        mesh=_MESH(),
        compiler_params=_SC_PARAMS,
        scratch_types=[
            pltpu.VMEM((NW, 8, 16), jnp.float32),
            pltpu.VMEM((8, BLK), jnp.float32),
            pltpu.VMEM((BLK, 8), jnp.float32),
            pltpu.VMEM((BLK,), jnp.int32),
            pltpu.VMEM_SHARED((np_rows, 8), jnp.float32),
        ],
    )


def _sc_c1(nblk_tile, np_rows, n_nodes):
    def body(src_h, dst_h, alpha_h, tmax_h, xp_h, d0_h, d1_h, zeros64_h,
             oparts_h, tmax_v, abuf, idx_s, idx_d, xprows, d0r, d1r, msg,
             shared_o):
        cid = lax.axis_index("c")
        sid = lax.axis_index("s")
        wid = _worker_id()
        iota = _iota16()
        pltpu.sync_copy(tmax_h, tmax_v)
        gvecs = _gmax_vecs(tmax_v, 8)

        @pl.when(sid == 0)
        def _():
            pltpu.sync_copy(zeros64_h, shared_o)
        plsc.subcore_barrier()

        def blk(b, _):
            tb = wid * nblk_tile + b
            base = tb * BLK
            pltpu.sync_copy(src_h.at[pl.ds(base, BLK)], idx_s)
            pltpu.sync_copy(dst_h.at[pl.ds(base, BLK)], idx_d)
            pltpu.sync_copy(alpha_h.at[tb], abuf)
            pltpu.sync_copy(xp_h.at[idx_s], xprows)
            pltpu.sync_copy(d0_h.at[idx_d], d0r)
            pltpu.sync_copy(d1_h.at[idx_d], d1r)
            for h in range(8):
                hh = jnp.full((16,), h, jnp.int32)
                for g in range(8):
                    ii = g * 16 + iota
                    a = abuf[h, pl.ds(g * 16, 16)]
                    ex = jnp.exp(a - gvecs[h])
                    den = (plsc.load_gather(d0r, [ii, hh])
                           + plsc.load_gather(d1r, [ii, hh]))
                    coef = ex / (den + 1e-16)
                    for c in range(8):
                        cc = jnp.full((16,), 8 * h + c, jnp.int32)
                        xv = plsc.load_gather(xprows, [ii, cc])
                        plsc.store_scatter(msg, [ii, cc], xv * coef)
            pltpu.sync_copy(msg, shared_o.at[idx_d], add=True)
            return 0

        lax.fori_loop(0, nblk_tile, blk, 0)
        plsc.subcore_barrier()

        @pl.when(sid == 0)
        def _():
            pltpu.sync_copy(shared_o, oparts_h.at[cid])

    return pl.kernel(
        body,
        out_type=[_f32((NC, np_rows, 64))],
        mesh=_MESH(),
        compiler_params=_SC_PARAMS,
        scratch_types=[
            pltpu.VMEM((NW, 8, 16), jnp.float32),
            pltpu.VMEM((8, BLK), jnp.float32),
            pltpu.VMEM((BLK,), jnp.int32),
            pltpu.VMEM((BLK,), jnp.int32),
            pltpu.VMEM((BLK, 64), jnp.float32),
            pltpu.VMEM((BLK, 8), jnp.float32),
            pltpu.VMEM((BLK, 8), jnp.float32),
            pltpu.VMEM((BLK, 64), jnp.float32),
            pltpu.VMEM_SHARED((np_rows, 64), jnp.float32),
        ],
    )


def _sc_a2(nblk_tile, np_rows):
    def body(src_h, dst_h, as_h, ad_h, alpha_h, tmax_h,
             as_v, ad_v, idx_s, idx_d, abuf, mbuf):
        wid = _worker_id()
        pltpu.sync_copy(as_h, as_v)
        pltpu.sync_copy(ad_h, ad_v)
        mbuf[:] = jnp.full((16,), NEG_INIT, jnp.float32)

        def blk(b, _):
            tb = wid * nblk_tile + b
            base = tb * BLK
            pltpu.sync_copy(src_h.at[pl.ds(base, BLK)], idx_s)
            pltpu.sync_copy(dst_h.at[pl.ds(base, BLK)], idx_d)
            mv = mbuf[:]
            for g in range(8):
                si = idx_s[pl.ds(g * 16, 16)]
                di = idx_d[pl.ds(g * 16, 16)]
                v = plsc.load_gather(as_v, [si]) + plsc.load_gather(ad_v, [di])
                a = jnp.maximum(v, 0.2 * v)
                mv = jnp.maximum(mv, a)
                abuf[pl.ds(g * 16, 16)] = a
            mbuf[:] = mv
            pltpu.sync_copy(abuf, alpha_h.at[tb])
            return 0

        lax.fori_loop(0, nblk_tile, blk, 0)
        pltpu.sync_copy(mbuf, tmax_h.at[wid])

    nblk = nblk_tile * NW
    return pl.kernel(
        body,
        out_type=[_f32((nblk, BLK)), _f32((NW, 16))],
        mesh=_MESH(),
        compiler_params=_SC_PARAMS,
        scratch_types=[
            pltpu.VMEM((np_rows,), jnp.float32),
            pltpu.VMEM((np_rows,), jnp.float32),
            pltpu.VMEM((BLK,), jnp.int32),
            pltpu.VMEM((BLK,), jnp.int32),
            pltpu.VMEM((BLK,), jnp.float32),
            pltpu.VMEM((16,), jnp.float32),
        ],
    )


def _sc_b2(nblk_tile, np_rows):
    def body(dst_h, alpha_h, tmax_h, zeros8_h, dparts_h,
             tmax_v, abuf, exbuf, idx_d, shared_d):
        cid = lax.axis_index("c")
        sid = lax.axis_index("s")
        wid = _worker_id()
        iota = _iota16()
        pltpu.sync_copy(tmax_h, tmax_v)
        gvec = _gmax_vec1(tmax_v)
        pltpu.sync_copy(zeros8_h.at[pl.ds(0, BLK)], exbuf)

        @pl.when(sid == 0)
        def _():
            pltpu.sync_copy(zeros8_h, shared_d)
        plsc.subcore_barrier()

        zz = jnp.full((16,), 0, jnp.int32)

        def blk(b, _):
            tb = wid * nblk_tile + b
            base = tb * BLK
            pltpu.sync_copy(dst_h.at[pl.ds(base, BLK)], idx_d)
            pltpu.sync_copy(alpha_h.at[tb], abuf)
            for g in range(8):
                a = abuf[pl.ds(g * 16, 16)]
                ex = jnp.exp(a - gvec)
                plsc.store_scatter(exbuf, [g * 16 + iota, zz], ex)
            pltpu.sync_copy(exbuf, shared_d.at[idx_d], add=True)
            return 0

        lax.fori_loop(0, nblk_tile, blk, 0)
        plsc.subcore_barrier()

        @pl.when(sid == 0)
        def _():
            pltpu.sync_copy(shared_d, dparts_h.at[cid])

    return pl.kernel(
        body,
        out_type=[_f32((NC, np_rows, 8))],
        mesh=_MESH(),
        compiler_params=_SC_PARAMS,
        scratch_types=[
            pltpu.VMEM((NW, 16), jnp.float32),
            pltpu.VMEM((BLK,), jnp.float32),
            pltpu.VMEM((BLK, 8), jnp.float32),
            pltpu.VMEM((BLK,), jnp.int32),
            pltpu.VMEM_SHARED((np_rows, 8), jnp.float32),
        ],
    )


def _sc_c2(nblk_tile, np_rows):
    def body(src_h, dst_h, alpha_h, tmax_h, xp_h, d0_h, d1_h, zeros8_h,
             oparts_h, tmax_v, abuf, idx_s, idx_d, xr, d0r, d1r, msg,
             shared_o):
        cid = lax.axis_index("c")
        sid = lax.axis_index("s")
        wid = _worker_id()
        iota = _iota16()
        pltpu.sync_copy(tmax_h, tmax_v)
        gvec = _gmax_vec1(tmax_v)

        @pl.when(sid == 0)
        def _():
            pltpu.sync_copy(zeros8_h, shared_o)
        plsc.subcore_barrier()

        zz = jnp.full((16,), 0, jnp.int32)

        def blk(b, _):
            tb = wid * nblk_tile + b
            base = tb * BLK
            pltpu.sync_copy(src_h.at[pl.ds(base, BLK)], idx_s)
            pltpu.sync_copy(dst_h.at[pl.ds(base, BLK)], idx_d)
            pltpu.sync_copy(alpha_h.at[tb], abuf)
            pltpu.sync_copy(xp_h.at[idx_s], xr)
            pltpu.sync_copy(d0_h.at[idx_d], d0r)
            pltpu.sync_copy(d1_h.at[idx_d], d1r)
            for g in range(8):
                ii = g * 16 + iota
                a = abuf[pl.ds(g * 16, 16)]
                ex = jnp.exp(a - gvec)
                den = (plsc.load_gather(d0r, [ii, zz])
                       + plsc.load_gather(d1r, [ii, zz]))
                coef = ex / (den + 1e-16)
                for c in range(8):
                    cc = jnp.full((16,), c, jnp.int32)
                    xv = plsc.load_gather(xr, [ii, cc])
                    plsc.store_scatter(msg, [ii, cc], xv * coef)
            pltpu.sync_copy(msg, shared_o.at[idx_d], add=True)
            return 0

        lax.fori_loop(0, nblk_tile, blk, 0)
        plsc.subcore_barrier()

        @pl.when(sid == 0)
        def _():
            pltpu.sync_copy(shared_o, oparts_h.at[cid])

    return pl.kernel(
        body,
        out_type=[_f32((NC, np_rows, 8))],
        mesh=_MESH(),
        compiler_params=_SC_PARAMS,
        scratch_types=[
            pltpu.VMEM((NW, 16), jnp.float32),
            pltpu.VMEM((BLK,), jnp.float32),
            pltpu.VMEM((BLK,), jnp.int32),
            pltpu.VMEM((BLK,), jnp.int32),
            pltpu.VMEM((BLK, 8), jnp.float32),
            pltpu.VMEM((BLK, 8), jnp.float32),
            pltpu.VMEM((BLK, 8), jnp.float32),
            pltpu.VMEM((BLK, 8), jnp.float32),
            pltpu.VMEM_SHARED((np_rows, 8), jnp.float32),
        ],
    )


def _tc1_body(x_ref, w_ref, asrc_ref, adst_ref, xp_ref, asad_ref):
    xp = jnp.dot(x_ref[:], w_ref[:], preferred_element_type=jnp.float32)
    xr = xp.reshape(xp.shape[0], 8, 8)
    a_s = (xr * asrc_ref[:][None]).sum(-1)
    a_d = (xr * adst_ref[:][None]).sum(-1)
    xp_ref[:] = xp
    asad_ref[:] = jnp.concatenate([a_s, a_d], axis=1)


def _tc2_body(p0_ref, p1_ref, b1_ref, w2_ref, a2s_ref, a2d_ref,
              xp2_ref, asad2_ref):
    v = p0_ref[:] + p1_ref[:] + b1_ref[:][None]
    h = jnp.where(v > 0, v, jnp.exp(v) - 1.0)
    xp2 = jnp.dot(h, w2_ref[:], preferred_element_type=jnp.float32)
    a_s = (xp2 * a2s_ref[:][None]).sum(-1, keepdims=True)
    a_d = (xp2 * a2d_ref[:][None]).sum(-1, keepdims=True)
    zero = jnp.zeros((xp2.shape[0], 6), jnp.float32)
    xp2_ref[:] = xp2
    asad2_ref[:] = jnp.concatenate([a_s, a_d, zero], axis=1)


def _tc3_body(p0_ref, p1_ref, b2_ref, out_ref):
    s = p0_ref[:] + p1_ref[:]
    x7 = s[:, :7] + b2_ref[:][None]
    m = jnp.max(x7, axis=1, keepdims=True)
    e = jnp.exp(x7 - m)
    lse = jnp.log(jnp.sum(e, axis=1, keepdims=True))
    out_ref[:] = x7 - m - lse


def kernel(x, edge_index, W1, att_src1, att_dst1, b1,
           W2, att_src2, att_dst2, b2):
    n, f_in = x.shape
    e = edge_index.shape[1]
    ne = e + n
    ep = -(-ne // (NW * BLK)) * (NW * BLK)
    nblk_tile = ep // (NW * BLK)
    np_rows = -(-(n + 1) // 16) * 16

    loops = jnp.arange(n, dtype=edge_index.dtype)
    pad = ep - ne
    srcp = jnp.concatenate(
        [edge_index[0], loops, jnp.zeros((pad,), edge_index.dtype)])
    dstp = jnp.concatenate(
        [edge_index[1], loops, jnp.full((pad,), n, edge_index.dtype)])
    zeros8 = jnp.zeros((np_rows, 8), jnp.float32)
    zeros64 = jnp.zeros((np_rows, 64), jnp.float32)
    a1s = att_src1.reshape(8, 8)
    a1d = att_dst1.reshape(8, 8)
    w2p = jnp.pad(W2, ((0, 0), (0, 1)))
    a2s = jnp.pad(att_src2.reshape(7), (0, 1))
    a2d = jnp.pad(att_dst2.reshape(7), (0, 1))

    rb = 400
    g1 = n // rb
    xp1, asad1 = pl.pallas_call(
        _tc1_body,
        grid=(g1,),
        in_specs=[
            pl.BlockSpec((rb, f_in), lambda i: (i, 0)),
            pl.BlockSpec((f_in, 64), lambda i: (0, 0)),
            pl.BlockSpec((8, 8), lambda i: (0, 0)),
            pl.BlockSpec((8, 8), lambda i: (0, 0)),
        ],
        out_specs=[
            pl.BlockSpec((rb, 64), lambda i: (i, 0)),
            pl.BlockSpec((rb, 16), lambda i: (i, 0)),
        ],
        out_shape=[_f32((n, 64)), _f32((n, 16))],
    )(x, W1, a1s, a1d)
    asad1f = jnp.concatenate(
        [asad1, jnp.zeros((np_rows - n, 16), jnp.float32)])

    alpha1, tmax1 = _sc_a1(nblk_tile, np_rows)(srcp, dstp, asad1f)
    (dparts1,) = _sc_b1(nblk_tile, np_rows)(dstp, alpha1, tmax1, zeros8)
    (oparts1,) = _sc_c1(nblk_tile, np_rows, n)(
        srcp, dstp, alpha1, tmax1, xp1, dparts1[0], dparts1[1], zeros64)

    xp2, asad2 = pl.pallas_call(
        _tc2_body,
        grid=(g1,),
        in_specs=[
            pl.BlockSpec((rb, 64), lambda i: (i, 0)),
            pl.BlockSpec((rb, 64), lambda i: (i, 0)),
            pl.BlockSpec((64,), lambda i: (0,)),
            pl.BlockSpec((64, 8), lambda i: (0, 0)),
            pl.BlockSpec((8,), lambda i: (0,)),
            pl.BlockSpec((8,), lambda i: (0,)),
        ],
        out_specs=[
            pl.BlockSpec((rb, 8), lambda i: (i, 0)),
            pl.BlockSpec((rb, 8), lambda i: (i, 0)),
        ],
        out_shape=[_f32((n, 8)), _f32((n, 8))],
    )(oparts1[0, :n], oparts1[1, :n], b1, w2p, a2s, a2d)
    as2f = jnp.concatenate([asad2[:, 0], jnp.zeros((np_rows - n,), jnp.float32)])
    ad2f = jnp.concatenate([asad2[:, 1], jnp.zeros((np_rows - n,), jnp.float32)])

    alpha2, tmax2 = _sc_a2(nblk_tile, np_rows)(srcp, dstp, as2f, ad2f)
    (dparts2,) = _sc_b2(nblk_tile, np_rows)(dstp, alpha2, tmax2, zeros8)
    (oparts2,) = _sc_c2(nblk_tile, np_rows)(
        srcp, dstp, alpha2, tmax2, xp2, dparts2[0], dparts2[1], zeros8)

    (out,) = pl.pallas_call(
        _tc3_body,
        grid=(g1,),
        in_specs=[
            pl.BlockSpec((rb, 8), lambda i: (i, 0)),
            pl.BlockSpec((rb, 8), lambda i: (i, 0)),
            pl.BlockSpec((7,), lambda i: (0,)),
        ],
        out_specs=[pl.BlockSpec((rb, 7), lambda i: (i, 0))],
        out_shape=[_f32((n, 7))],
    )(oparts2[0, :n], oparts2[1, :n], b2)
    return out


# speedup vs baseline: 29.5314x; 1.6100x over previous
import functools

import jax
import jax.numpy as jnp
from jax import lax
from jax.experimental import pallas as pl
from jax.experimental.pallas import tpu as pltpu
from jax.experimental.pallas import tpu_sc as plsc

NC, NS = 2, 16
NW = NC * NS
BLK = 128
NEG_INIT = -1e30

_SC_PARAMS = pltpu.CompilerParams(
    needs_layout_passes=False, use_tc_tiling_on_sc=False)

_MESH = functools.partial(
    plsc.VectorSubcoreMesh,
    core_axis_name="c", subcore_axis_name="s",
    num_cores=NC, num_subcores=NS,
)


def _f32(shape):
    return jax.ShapeDtypeStruct(shape, jnp.float32)


def _worker_id():
    return lax.axis_index("s") * NC + lax.axis_index("c")


def _iota16():
    return lax.iota(jnp.int32, 16)


def _wait(src, dst, sem):
    pltpu.make_async_copy(src, dst, sem).wait()


def _gmax_vecs(tmax_ref, heads):
    out = []
    for h in range(heads):
        m = tmax_ref[0, h, :]
        for t in range(1, NW):
            m = jnp.maximum(m, tmax_ref[t, h, :])
        out.append(jnp.full((16,), jnp.max(m), jnp.float32))
    return out


def _gmax_vec1(tmax_ref):
    m = tmax_ref[0, :]
    for t in range(1, NW):
        m = jnp.maximum(m, tmax_ref[t, :])
    return jnp.full((16,), jnp.max(m), jnp.float32)


def _pipeline(nblk_tile, tbase, issue_in, wait_in, compute, issue_out,
              wait_out):
    issue_in(0, tbase)

    def step(i, _):
        j = i * 2
        for p in (0, 1):
            b = j + p
            nxt = b + 1

            if p == 0:
                issue_in(1, tbase + nxt)
            else:
                @pl.when(nxt < nblk_tile)
                def _():
                    issue_in(0, tbase + nxt)
            wait_in(p)

            @pl.when(b >= 2)
            def _():
                wait_out(p)
            compute(p, tbase + b)
            issue_out(p, tbase + b)
        return 0

    lax.fori_loop(0, nblk_tile // 2, step, 0)
    wait_out(0)
    wait_out(1)


def _sc_a1(nblk_tile, np_rows):
    def body(src_h, dst_h, asad_h, alpha_h, tmax_h,
             isa, ida, rs0, rs1, rd0, rd1, ab0, ab1, mbuf,
             sin0, sin1, so0, so1):
        wid = _worker_id()
        iota = _iota16()
        tbase = wid * nblk_tile
        pltpu.sync_copy(src_h.at[pl.ds(tbase, nblk_tile)], isa)
        pltpu.sync_copy(dst_h.at[pl.ds(tbase, nblk_tile)], ida)
        rs = (rs0, rs1)
        rd = (rd0, rd1)
        ab = (ab0, ab1)
        sin = (sin0, sin1)
        so = (so0, so1)
        for h in range(8):
            mbuf[h, :] = jnp.full((16,), NEG_INIT, jnp.float32)

        def issue_in(p, tb):
            b = tb - tbase
            pltpu.async_copy(asad_h.at[isa.at[b]], rs[p], sin[p])
            pltpu.async_copy(asad_h.at[ida.at[b]], rd[p], sin[p])

        def wait_in(p):
            _wait(asad_h.at[isa.at[0]], rs[p], sin[p])
            _wait(asad_h.at[ida.at[0]], rd[p], sin[p])

        def compute(p, tb):
            for h in range(8):
                hh_s = jnp.full((16,), h, jnp.int32)
                hh_d = jnp.full((16,), h + 8, jnp.int32)
                mv = mbuf[h, :]
                for g in range(8):
                    ii = g * 16 + iota
                    sv = plsc.load_gather(rs[p], [ii, hh_s])
                    dv = plsc.load_gather(rd[p], [ii, hh_d])
                    v = sv + dv
                    a = jnp.maximum(v, 0.2 * v)
                    mv = jnp.maximum(mv, a)
                    ab[p][h, pl.ds(g * 16, 16)] = a
                mbuf[h, :] = mv

        def issue_out(p, tb):
            pltpu.async_copy(ab[p], alpha_h.at[tb], so[p])

        def wait_out(p):
            _wait(ab[p], alpha_h.at[0], so[p])

        _pipeline(nblk_tile, tbase, issue_in, wait_in, compute, issue_out,
                  wait_out)
        pltpu.sync_copy(mbuf, tmax_h.at[wid])

    nblk = nblk_tile * NW
    return pl.kernel(
        body,
        out_type=[_f32((nblk, 8, BLK)), _f32((NW, 8, 16))],
        mesh=_MESH(),
        compiler_params=_SC_PARAMS,
        scratch_types=[
            pltpu.VMEM((nblk_tile, BLK), jnp.int32),
            pltpu.VMEM((nblk_tile, BLK), jnp.int32),
            pltpu.VMEM((BLK, 16), jnp.float32),
            pltpu.VMEM((BLK, 16), jnp.float32),
            pltpu.VMEM((BLK, 16), jnp.float32),
            pltpu.VMEM((BLK, 16), jnp.float32),
            pltpu.VMEM((8, BLK), jnp.float32),
            pltpu.VMEM((8, BLK), jnp.float32),
            pltpu.VMEM((8, 16), jnp.float32),
            pltpu.SemaphoreType.DMA,
            pltpu.SemaphoreType.DMA,
            pltpu.SemaphoreType.DMA,
            pltpu.SemaphoreType.DMA,
        ],
    )


def _sc_b1(nblk_tile, np_rows):
    def body(dst_h, alpha_h, tmax_h, zeros8_h, dparts_h,
             ida, tmax_v, ab0, ab1, ex0, ex1, shared_d,
             sin0, sin1, so0, so1):
        cid = lax.axis_index("c")
        sid = lax.axis_index("s")
        wid = _worker_id()
        iota = _iota16()
        tbase = wid * nblk_tile
        pltpu.sync_copy(dst_h.at[pl.ds(tbase, nblk_tile)], ida)
        pltpu.sync_copy(tmax_h, tmax_v)
        gvecs = _gmax_vecs(tmax_v, 8)
        ab = (ab0, ab1)
        ex = (ex0, ex1)
        sin = (sin0, sin1)
        so = (so0, so1)

        @pl.when(sid == 0)
        def _():
            pltpu.sync_copy(zeros8_h, shared_d)
        plsc.subcore_barrier()

        def issue_in(p, tb):
            pltpu.async_copy(alpha_h.at[tb], ab[p], sin[p])

        def wait_in(p):
            _wait(alpha_h.at[0], ab[p], sin[p])

        def compute(p, tb):
            for h in range(8):
                hh = jnp.full((16,), h, jnp.int32)
                for g in range(8):
                    a = ab[p][h, pl.ds(g * 16, 16)]
                    e = jnp.exp(a - gvecs[h])
                    plsc.store_scatter(ex[p], [g * 16 + iota, hh], e)

        def issue_out(p, tb):
            pltpu.async_copy(ex[p], shared_d.at[ida.at[tb - tbase]], so[p],
                             add=True)

        def wait_out(p):
            _wait(ex[p], shared_d.at[ida.at[0]], so[p])

        _pipeline(nblk_tile, tbase, issue_in, wait_in, compute, issue_out,
                  wait_out)
        plsc.subcore_barrier()

        @pl.when(sid == 0)
        def _():
            pltpu.sync_copy(shared_d, dparts_h.at[cid])

    return pl.kernel(
        body,
        out_type=[_f32((NC, np_rows, 8))],
        mesh=_MESH(),
        compiler_params=_SC_PARAMS,
        scratch_types=[
            pltpu.VMEM((nblk_tile, BLK), jnp.int32),
            pltpu.VMEM((NW, 8, 16), jnp.float32),
            pltpu.VMEM((8, BLK), jnp.float32),
            pltpu.VMEM((8, BLK), jnp.float32),
            pltpu.VMEM((BLK, 8), jnp.float32),
            pltpu.VMEM((BLK, 8), jnp.float32),
            pltpu.VMEM_SHARED((np_rows, 8), jnp.float32),
            pltpu.SemaphoreType.DMA,
            pltpu.SemaphoreType.DMA,
            pltpu.SemaphoreType.DMA,
            pltpu.SemaphoreType.DMA,
        ],
    )


def _sc_c1(nblk_tile, np_rows):
    def body(src_h, dst_h, alpha_h, tmax_h, xp_h, d0_h, d1_h, zeros64_h,
             oparts_h,
             isa, ida, tmax_v, ab0, ab1, xr0, xr1, d0r0, d0r1, d1r0, d1r1,
             mg0, mg1, shared_o, sin0, sin1, so0, so1):
        cid = lax.axis_index("c")
        sid = lax.axis_index("s")
        wid = _worker_id()
        iota = _iota16()
        tbase = wid * nblk_tile
        pltpu.sync_copy(src_h.at[pl.ds(tbase, nblk_tile)], isa)
        pltpu.sync_copy(dst_h.at[pl.ds(tbase, nblk_tile)], ida)
        pltpu.sync_copy(tmax_h, tmax_v)
        gvecs = _gmax_vecs(tmax_v, 8)
        ab = (ab0, ab1)
        xr = (xr0, xr1)
        d0r = (d0r0, d0r1)
        d1r = (d1r0, d1r1)
        mg = (mg0, mg1)
        sin = (sin0, sin1)
        so = (so0, so1)

        @pl.when(sid == 0)
        def _():
            pltpu.sync_copy(zeros64_h, shared_o)
        plsc.subcore_barrier()

        def issue_in(p, tb):
            b = tb - tbase
            pltpu.async_copy(alpha_h.at[tb], ab[p], sin[p])
            pltpu.async_copy(xp_h.at[isa.at[b]], xr[p], sin[p])
            pltpu.async_copy(d0_h.at[ida.at[b]], d0r[p], sin[p])
            pltpu.async_copy(d1_h.at[ida.at[b]], d1r[p], sin[p])

        def wait_in(p):
            _wait(alpha_h.at[0], ab[p], sin[p])
            _wait(xp_h.at[isa.at[0]], xr[p], sin[p])
            _wait(d0_h.at[ida.at[0]], d0r[p], sin[p])
            _wait(d1_h.at[ida.at[0]], d1r[p], sin[p])

        def compute(p, tb):
            for h in range(8):
                hh = jnp.full((16,), h, jnp.int32)
                for g in range(8):
                    ii = g * 16 + iota
                    a = ab[p][h, pl.ds(g * 16, 16)]
                    e = jnp.exp(a - gvecs[h])
                    den = (plsc.load_gather(d0r[p], [ii, hh])
                           + plsc.load_gather(d1r[p], [ii, hh]))
                    coef = e / (den + 1e-16)
                    for c in range(8):
                        cc = jnp.full((16,), 8 * h + c, jnp.int32)
                        xv = plsc.load_gather(xr[p], [ii, cc])
                        plsc.store_scatter(mg[p], [ii, cc], xv * coef)

        def issue_out(p, tb):
            pltpu.async_copy(mg[p], shared_o.at[ida.at[tb - tbase]], so[p],
                             add=True)

        def wait_out(p):
            _wait(mg[p], shared_o.at[ida.at[0]], so[p])

        _pipeline(nblk_tile, tbase, issue_in, wait_in, compute, issue_out,
                  wait_out)
        plsc.subcore_barrier()

        @pl.when(sid == 0)
        def _():
            pltpu.sync_copy(shared_o, oparts_h.at[cid])

    return pl.kernel(
        body,
        out_type=[_f32((NC, np_rows, 64))],
        mesh=_MESH(),
        compiler_params=_SC_PARAMS,
        scratch_types=[
            pltpu.VMEM((nblk_tile, BLK), jnp.int32),
            pltpu.VMEM((nblk_tile, BLK), jnp.int32),
            pltpu.VMEM((NW, 8, 16), jnp.float32),
            pltpu.VMEM((8, BLK), jnp.float32),
            pltpu.VMEM((8, BLK), jnp.float32),
            pltpu.VMEM((BLK, 64), jnp.float32),
            pltpu.VMEM((BLK, 64), jnp.float32),
            pltpu.VMEM((BLK, 8), jnp.float32),
            pltpu.VMEM((BLK, 8), jnp.float32),
            pltpu.VMEM((BLK, 8), jnp.float32),
            pltpu.VMEM((BLK, 8), jnp.float32),
            pltpu.VMEM((BLK, 64), jnp.float32),
            pltpu.VMEM((BLK, 64), jnp.float32),
            pltpu.VMEM_SHARED((np_rows, 64), jnp.float32),
            pltpu.SemaphoreType.DMA,
            pltpu.SemaphoreType.DMA,
            pltpu.SemaphoreType.DMA,
            pltpu.SemaphoreType.DMA,
        ],
    )


def _sc_a2(nblk_tile, np_rows):
    def body(src_h, dst_h, as_h, ad_h, alpha_h, tmax_h,
             as_v, ad_v, isa, ida, ab0, ab1, mbuf, so0, so1):
        wid = _worker_id()
        tbase = wid * nblk_tile
        pltpu.sync_copy(as_h, as_v)
        pltpu.sync_copy(ad_h, ad_v)
        pltpu.sync_copy(src_h.at[pl.ds(tbase, nblk_tile)], isa)
        pltpu.sync_copy(dst_h.at[pl.ds(tbase, nblk_tile)], ida)
        mbuf[:] = jnp.full((16,), NEG_INIT, jnp.float32)
        ab = (ab0, ab1)
        so = (so0, so1)

        def issue_in(p, tb):
            pass

        def wait_in(p):
            pass

        def compute(p, tb):
            b = tb - tbase
            mv = mbuf[:]
            for g in range(8):
                si = isa[b, pl.ds(g * 16, 16)]
                di = ida[b, pl.ds(g * 16, 16)]
                v = plsc.load_gather(as_v, [si]) + plsc.load_gather(ad_v, [di])
                a = jnp.maximum(v, 0.2 * v)
                mv = jnp.maximum(mv, a)
                ab[p][pl.ds(g * 16, 16)] = a
            mbuf[:] = mv

        def issue_out(p, tb):
            pltpu.async_copy(ab[p], alpha_h.at[tb], so[p])

        def wait_out(p):
            _wait(ab[p], alpha_h.at[0], so[p])

        _pipeline(nblk_tile, tbase, issue_in, wait_in, compute, issue_out,
                  wait_out)
        pltpu.sync_copy(mbuf, tmax_h.at[wid])

    nblk = nblk_tile * NW
    return pl.kernel(
        body,
        out_type=[_f32((nblk, BLK)), _f32((NW, 16))],
        mesh=_MESH(),
        compiler_params=_SC_PARAMS,
        scratch_types=[
            pltpu.VMEM((np_rows,), jnp.float32),
            pltpu.VMEM((np_rows,), jnp.float32),
            pltpu.VMEM((nblk_tile, BLK), jnp.int32),
            pltpu.VMEM((nblk_tile, BLK), jnp.int32),
            pltpu.VMEM((BLK,), jnp.float32),
            pltpu.VMEM((BLK,), jnp.float32),
            pltpu.VMEM((16,), jnp.float32),
            pltpu.SemaphoreType.DMA,
            pltpu.SemaphoreType.DMA,
        ],
    )


def _sc_b2(nblk_tile, np_rows):
    def body(dst_h, alpha_h, tmax_h, zeros8_h, dparts_h,
             ida, tmax_v, ab0, ab1, ex0, ex1, shared_d,
             sin0, sin1, so0, so1):
        cid = lax.axis_index("c")
        sid = lax.axis_index("s")
        wid = _worker_id()
        iota = _iota16()
        tbase = wid * nblk_tile
        pltpu.sync_copy(dst_h.at[pl.ds(tbase, nblk_tile)], ida)
        pltpu.sync_copy(tmax_h, tmax_v)
        gvec = _gmax_vec1(tmax_v)
        pltpu.sync_copy(zeros8_h.at[pl.ds(0, BLK)], ex0)
        pltpu.sync_copy(zeros8_h.at[pl.ds(0, BLK)], ex1)
        ab = (ab0, ab1)
        ex = (ex0, ex1)
        sin = (sin0, sin1)
        so = (so0, so1)

        @pl.when(sid == 0)
        def _():
            pltpu.sync_copy(zeros8_h, shared_d)
        plsc.subcore_barrier()

        zz = jnp.full((16,), 0, jnp.int32)

        def issue_in(p, tb):
            pltpu.async_copy(alpha_h.at[tb], ab[p], sin[p])

        def wait_in(p):
            _wait(alpha_h.at[0], ab[p], sin[p])

        def compute(p, tb):
            for g in range(8):
                a = ab[p][pl.ds(g * 16, 16)]
                e = jnp.exp(a - gvec)
                plsc.store_scatter(ex[p], [g * 16 + iota, zz], e)

        def issue_out(p, tb):
            pltpu.async_copy(ex[p], shared_d.at[ida.at[tb - tbase]], so[p],
                             add=True)

        def wait_out(p):
            _wait(ex[p], shared_d.at[ida.at[0]], so[p])

        _pipeline(nblk_tile, tbase, issue_in, wait_in, compute, issue_out,
                  wait_out)
        plsc.subcore_barrier()

        @pl.when(sid == 0)
        def _():
            pltpu.sync_copy(shared_d, dparts_h.at[cid])

    return pl.kernel(
        body,
        out_type=[_f32((NC, np_rows, 8))],
        mesh=_MESH(),
        compiler_params=_SC_PARAMS,
        scratch_types=[
            pltpu.VMEM((nblk_tile, BLK), jnp.int32),
            pltpu.VMEM((NW, 16), jnp.float32),
            pltpu.VMEM((BLK,), jnp.float32),
            pltpu.VMEM((BLK,), jnp.float32),
            pltpu.VMEM((BLK, 8), jnp.float32),
            pltpu.VMEM((BLK, 8), jnp.float32),
            pltpu.VMEM_SHARED((np_rows, 8), jnp.float32),
            pltpu.SemaphoreType.DMA,
            pltpu.SemaphoreType.DMA,
            pltpu.SemaphoreType.DMA,
            pltpu.SemaphoreType.DMA,
        ],
    )


def _sc_c2(nblk_tile, np_rows):
    def body(src_h, dst_h, alpha_h, tmax_h, xp_h, d0_h, d1_h, zeros8_h,
             oparts_h,
             isa, ida, tmax_v, ab0, ab1, xr0, xr1, d0r0, d0r1, d1r0, d1r1,
             mg0, mg1, shared_o, sin0, sin1, so0, so1):
        cid = lax.axis_index("c")
        sid = lax.axis_index("s")
        wid = _worker_id()
        iota = _iota16()
        tbase = wid * nblk_tile
        pltpu.sync_copy(src_h.at[pl.ds(tbase, nblk_tile)], isa)
        pltpu.sync_copy(dst_h.at[pl.ds(tbase, nblk_tile)], ida)
        pltpu.sync_copy(tmax_h, tmax_v)
        gvec = _gmax_vec1(tmax_v)
        ab = (ab0, ab1)
        xr = (xr0, xr1)
        d0r = (d0r0, d0r1)
        d1r = (d1r0, d1r1)
        mg = (mg0, mg1)
        sin = (sin0, sin1)
        so = (so0, so1)

        @pl.when(sid == 0)
        def _():
            pltpu.sync_copy(zeros8_h, shared_o)
        plsc.subcore_barrier()

        zz = jnp.full((16,), 0, jnp.int32)

        def issue_in(p, tb):
            b = tb - tbase
            pltpu.async_copy(alpha_h.at[tb], ab[p], sin[p])
            pltpu.async_copy(xp_h.at[isa.at[b]], xr[p], sin[p])
            pltpu.async_copy(d0_h.at[ida.at[b]], d0r[p], sin[p])
            pltpu.async_copy(d1_h.at[ida.at[b]], d1r[p], sin[p])

        def wait_in(p):
            _wait(alpha_h.at[0], ab[p], sin[p])
            _wait(xp_h.at[isa.at[0]], xr[p], sin[p])
            _wait(d0_h.at[ida.at[0]], d0r[p], sin[p])
            _wait(d1_h.at[ida.at[0]], d1r[p], sin[p])

        def compute(p, tb):
            for g in range(8):
                ii = g * 16 + iota
                a = ab[p][pl.ds(g * 16, 16)]
                e = jnp.exp(a - gvec)
                den = (plsc.load_gather(d0r[p], [ii, zz])
                       + plsc.load_gather(d1r[p], [ii, zz]))
                coef = e / (den + 1e-16)
                for c in range(8):
                    cc = jnp.full((16,), c, jnp.int32)
                    xv = plsc.load_gather(xr[p], [ii, cc])
                    plsc.store_scatter(mg[p], [ii, cc], xv * coef)

        def issue_out(p, tb):
            pltpu.async_copy(mg[p], shared_o.at[ida.at[tb - tbase]], so[p],
                             add=True)

        def wait_out(p):
            _wait(mg[p], shared_o.at[ida.at[0]], so[p])

        _pipeline(nblk_tile, tbase, issue_in, wait_in, compute, issue_out,
                  wait_out)
        plsc.subcore_barrier()

        @pl.when(sid == 0)
        def _():
            pltpu.sync_copy(shared_o, oparts_h.at[cid])

    return pl.kernel(
        body,
        out_type=[_f32((NC, np_rows, 8))],
        mesh=_MESH(),
        compiler_params=_SC_PARAMS,
        scratch_types=[
            pltpu.VMEM((nblk_tile, BLK), jnp.int32),
            pltpu.VMEM((nblk_tile, BLK), jnp.int32),
            pltpu.VMEM((NW, 16), jnp.float32),
            pltpu.VMEM((BLK,), jnp.float32),
            pltpu.VMEM((BLK,), jnp.float32),
            pltpu.VMEM((BLK, 8), jnp.float32),
            pltpu.VMEM((BLK, 8), jnp.float32),
            pltpu.VMEM((BLK, 8), jnp.float32),
            pltpu.VMEM((BLK, 8), jnp.float32),
            pltpu.VMEM((BLK, 8), jnp.float32),
            pltpu.VMEM((BLK, 8), jnp.float32),
            pltpu.VMEM((BLK, 8), jnp.float32),
            pltpu.VMEM((BLK, 8), jnp.float32),
            pltpu.VMEM_SHARED((np_rows, 8), jnp.float32),
            pltpu.SemaphoreType.DMA,
            pltpu.SemaphoreType.DMA,
            pltpu.SemaphoreType.DMA,
            pltpu.SemaphoreType.DMA,
        ],
    )


def _tc1_body(x_ref, w_ref, asrc_ref, adst_ref, xp_ref, asad_ref):
    xp = jnp.dot(x_ref[:], w_ref[:], preferred_element_type=jnp.float32)
    xr = xp.reshape(xp.shape[0], 8, 8)
    a_s = (xr * asrc_ref[:][None]).sum(-1)
    a_d = (xr * adst_ref[:][None]).sum(-1)
    xp_ref[:] = xp
    asad_ref[:] = jnp.concatenate([a_s, a_d], axis=1)


def _tc2_body(p0_ref, p1_ref, b1_ref, w2_ref, a2s_ref, a2d_ref,
              xp2_ref, asad2_ref):
    v = p0_ref[:] + p1_ref[:] + b1_ref[:][None]
    h = jnp.where(v > 0, v, jnp.exp(v) - 1.0)
    xp2 = jnp.dot(h, w2_ref[:], preferred_element_type=jnp.float32)
    a_s = (xp2 * a2s_ref[:][None]).sum(-1, keepdims=True)
    a_d = (xp2 * a2d_ref[:][None]).sum(-1, keepdims=True)
    zero = jnp.zeros((xp2.shape[0], 6), jnp.float32)
    xp2_ref[:] = xp2
    asad2_ref[:] = jnp.concatenate([a_s, a_d, zero], axis=1)


def _tc3_body(p0_ref, p1_ref, b2_ref, out_ref):
    s = p0_ref[:] + p1_ref[:]
    x7 = s[:, :7] + b2_ref[:][None]
    m = jnp.max(x7, axis=1, keepdims=True)
    e = jnp.exp(x7 - m)
    lse = jnp.log(jnp.sum(e, axis=1, keepdims=True))
    out_ref[:] = x7 - m - lse


def kernel(x, edge_index, W1, att_src1, att_dst1, b1,
           W2, att_src2, att_dst2, b2):
    n, f_in = x.shape
    e = edge_index.shape[1]
    ne = e + n
    ep = -(-ne // (NW * BLK)) * (NW * BLK)
    nblk = ep // BLK
    nblk_tile = nblk // NW
    np_rows = -(-(n + 1) // 16) * 16

    loops = jnp.arange(n, dtype=edge_index.dtype)
    pad = ep - ne
    srcp = jnp.concatenate(
        [edge_index[0], loops, jnp.zeros((pad,), edge_index.dtype)]
    ).reshape(nblk, BLK)
    dstp = jnp.concatenate(
        [edge_index[1], loops, jnp.full((pad,), n, edge_index.dtype)]
    ).reshape(nblk, BLK)
    zeros8 = jnp.zeros((np_rows, 8), jnp.float32)
    zeros64 = jnp.zeros((np_rows, 64), jnp.float32)
    a1s = att_src1.reshape(8, 8)
    a1d = att_dst1.reshape(8, 8)
    w2p = jnp.pad(W2, ((0, 0), (0, 1)))
    a2s = jnp.pad(att_src2.reshape(7), (0, 1))
    a2d = jnp.pad(att_dst2.reshape(7), (0, 1))

    rb = 400
    g1 = n // rb
    xp1, asad1 = pl.pallas_call(
        _tc1_body,
        grid=(g1,),
        in_specs=[
            pl.BlockSpec((rb, f_in), lambda i: (i, 0)),
            pl.BlockSpec((f_in, 64), lambda i: (0, 0)),
            pl.BlockSpec((8, 8), lambda i: (0, 0)),
            pl.BlockSpec((8, 8), lambda i: (0, 0)),
        ],
        out_specs=[
            pl.BlockSpec((rb, 64), lambda i: (i, 0)),
            pl.BlockSpec((rb, 16), lambda i: (i, 0)),
        ],
        out_shape=[_f32((n, 64)), _f32((n, 16))],
    )(x, W1, a1s, a1d)
    asad1f = jnp.concatenate(
        [asad1, jnp.zeros((np_rows - n, 16), jnp.float32)])

    alpha1, tmax1 = _sc_a1(nblk_tile, np_rows)(srcp, dstp, asad1f)
    (dparts1,) = _sc_b1(nblk_tile, np_rows)(dstp, alpha1, tmax1, zeros8)
    (oparts1,) = _sc_c1(nblk_tile, np_rows)(
        srcp, dstp, alpha1, tmax1, xp1, dparts1[0], dparts1[1], zeros64)

    xp2, asad2 = pl.pallas_call(
        _tc2_body,
        grid=(g1,),
        in_specs=[
            pl.BlockSpec((rb, 64), lambda i: (i, 0)),
            pl.BlockSpec((rb, 64), lambda i: (i, 0)),
            pl.BlockSpec((64,), lambda i: (0,)),
            pl.BlockSpec((64, 8), lambda i: (0, 0)),
            pl.BlockSpec((8,), lambda i: (0,)),
            pl.BlockSpec((8,), lambda i: (0,)),
        ],
        out_specs=[
            pl.BlockSpec((rb, 8), lambda i: (i, 0)),
            pl.BlockSpec((rb, 8), lambda i: (i, 0)),
        ],
        out_shape=[_f32((n, 8)), _f32((n, 8))],
    )(oparts1[0, :n], oparts1[1, :n], b1, w2p, a2s, a2d)
    as2f = jnp.concatenate([asad2[:, 0], jnp.zeros((np_rows - n,), jnp.float32)])
    ad2f = jnp.concatenate([asad2[:, 1], jnp.zeros((np_rows - n,), jnp.float32)])

    alpha2, tmax2 = _sc_a2(nblk_tile, np_rows)(srcp, dstp, as2f, ad2f)
    (dparts2,) = _sc_b2(nblk_tile, np_rows)(dstp, alpha2, tmax2, zeros8)
    (oparts2,) = _sc_c2(nblk_tile, np_rows)(
        srcp, dstp, alpha2, tmax2, xp2, dparts2[0], dparts2[1], zeros8)

    (out,) = pl.pallas_call(
        _tc3_body,
        grid=(g1,),
        in_specs=[
            pl.BlockSpec((rb, 8), lambda i: (i, 0)),
            pl.BlockSpec((rb, 8), lambda i: (i, 0)),
            pl.BlockSpec((7,), lambda i: (0,)),
        ],
        out_specs=[pl.BlockSpec((rb, 7), lambda i: (i, 0))],
        out_shape=[_f32((n, 7))],
    )(oparts2[0, :n], oparts2[1, :n], b2)
    return out


# speedup vs baseline: 32.5213x; 1.1012x over previous
import functools

import jax
import jax.numpy as jnp
from jax import lax
from jax.experimental import pallas as pl
from jax.experimental.pallas import tpu as pltpu
from jax.experimental.pallas import tpu_sc as plsc

NC, NS = 2, 16
NW = NC * NS
BLK = 128
NEG_INIT = -1e30

_SC_PARAMS = pltpu.CompilerParams(
    needs_layout_passes=False, use_tc_tiling_on_sc=False)

_MESH = functools.partial(
    plsc.VectorSubcoreMesh,
    core_axis_name="c", subcore_axis_name="s",
    num_cores=NC, num_subcores=NS,
)


def _f32(shape):
    return jax.ShapeDtypeStruct(shape, jnp.float32)


def _worker_id():
    return lax.axis_index("s") * NC + lax.axis_index("c")


def _iota16():
    return lax.iota(jnp.int32, 16)


def _wait(src, dst, sem):
    pltpu.make_async_copy(src, dst, sem).wait()


def _gmax_vecs(tmax_ref, heads):
    out = []
    for h in range(heads):
        m = tmax_ref[0, h, :]
        for t in range(1, NW):
            m = jnp.maximum(m, tmax_ref[t, h, :])
        out.append(jnp.full((16,), jnp.max(m), jnp.float32))
    return out


def _gmax_vec1(tmax_ref):
    m = tmax_ref[0, :]
    for t in range(1, NW):
        m = jnp.maximum(m, tmax_ref[t, :])
    return jnp.full((16,), jnp.max(m), jnp.float32)


def _pipeline(nblk_tile, tbase, issue_in, wait_in, compute, issue_out,
              wait_out):
    issue_in(0, tbase)

    def step(i, _):
        j = i * 2
        for p in (0, 1):
            b = j + p
            nxt = b + 1

            if p == 0:
                issue_in(1, tbase + nxt)
            else:
                @pl.when(nxt < nblk_tile)
                def _():
                    issue_in(0, tbase + nxt)
            wait_in(p)

            @pl.when(b >= 2)
            def _():
                wait_out(p)
            compute(p, tbase + b)
            issue_out(p, tbase + b)
        return 0

    lax.fori_loop(0, nblk_tile // 2, step, 0)
    wait_out(0)
    wait_out(1)


def _sc_a1(nblk_tile, np_rows):
    def body(src_h, dst_h, asad_h, alpha_h, tmax_h,
             isa, ida, rs0, rs1, rd0, rd1, ab0, ab1, mbuf,
             sin0, sin1, so0, so1):
        wid = _worker_id()
        iota = _iota16()
        tbase = wid * nblk_tile
        pltpu.sync_copy(src_h.at[pl.ds(tbase, nblk_tile)], isa)
        pltpu.sync_copy(dst_h.at[pl.ds(tbase, nblk_tile)], ida)
        rs = (rs0, rs1)
        rd = (rd0, rd1)
        ab = (ab0, ab1)
        sin = (sin0, sin1)
        so = (so0, so1)
        for h in range(8):
            mbuf[h, :] = jnp.full((16,), NEG_INIT, jnp.float32)

        def issue_in(p, tb):
            b = tb - tbase
            pltpu.async_copy(asad_h.at[isa.at[b]], rs[p], sin[p])
            pltpu.async_copy(asad_h.at[ida.at[b]], rd[p], sin[p])

        def wait_in(p):
            _wait(asad_h.at[isa.at[0]], rs[p], sin[p])
            _wait(asad_h.at[ida.at[0]], rd[p], sin[p])

        def compute(p, tb):
            for h in range(8):
                hh_s = jnp.full((16,), h, jnp.int32)
                hh_d = jnp.full((16,), h + 8, jnp.int32)
                mv = mbuf[h, :]
                for g in range(8):
                    ii = g * 16 + iota
                    sv = plsc.load_gather(rs[p], [ii, hh_s])
                    dv = plsc.load_gather(rd[p], [ii, hh_d])
                    v = sv + dv
                    a = jnp.maximum(v, 0.2 * v)
                    mv = jnp.maximum(mv, a)
                    ab[p][h, pl.ds(g * 16, 16)] = a
                mbuf[h, :] = mv

        def issue_out(p, tb):
            pltpu.async_copy(ab[p], alpha_h.at[tb], so[p])

        def wait_out(p):
            _wait(ab[p], alpha_h.at[0], so[p])

        _pipeline(nblk_tile, tbase, issue_in, wait_in, compute, issue_out,
                  wait_out)
        pltpu.sync_copy(mbuf, tmax_h.at[wid])

    nblk = nblk_tile * NW
    return pl.kernel(
        body,
        out_type=[_f32((nblk, 8, BLK)), _f32((NW, 8, 16))],
        mesh=_MESH(),
        compiler_params=_SC_PARAMS,
        scratch_types=[
            pltpu.VMEM((nblk_tile, BLK), jnp.int32),
            pltpu.VMEM((nblk_tile, BLK), jnp.int32),
            pltpu.VMEM((BLK, 16), jnp.float32),
            pltpu.VMEM((BLK, 16), jnp.float32),
            pltpu.VMEM((BLK, 16), jnp.float32),
            pltpu.VMEM((BLK, 16), jnp.float32),
            pltpu.VMEM((8, BLK), jnp.float32),
            pltpu.VMEM((8, BLK), jnp.float32),
            pltpu.VMEM((8, 16), jnp.float32),
            pltpu.SemaphoreType.DMA,
            pltpu.SemaphoreType.DMA,
            pltpu.SemaphoreType.DMA,
            pltpu.SemaphoreType.DMA,
        ],
    )


def _sc_bc1(nblk_tile, np_rows):
    def body(src_h, dst_h, alpha_h, tmax_h, xp_h, zeros8_h, zeros64_h,
             dparts_h, oparts_h,
             isa, ida, tmax_v, ab0, ab1, xr0, xr1, ex0, ex1,
             mg0, mg1, shared_d, shared_o, sin0, sin1, so0, so1):
        cid = lax.axis_index("c")
        sid = lax.axis_index("s")
        wid = _worker_id()
        iota = _iota16()
        tbase = wid * nblk_tile
        pltpu.sync_copy(src_h.at[pl.ds(tbase, nblk_tile)], isa)
        pltpu.sync_copy(dst_h.at[pl.ds(tbase, nblk_tile)], ida)
        pltpu.sync_copy(tmax_h, tmax_v)
        gvecs = _gmax_vecs(tmax_v, 8)
        ab = (ab0, ab1)
        xr = (xr0, xr1)
        ex = (ex0, ex1)
        mg = (mg0, mg1)
        sin = (sin0, sin1)
        so = (so0, so1)

        @pl.when(sid == 0)
        def _():
            pltpu.sync_copy(zeros8_h, shared_d)
            pltpu.sync_copy(zeros64_h, shared_o)
        plsc.subcore_barrier()

        def issue_in(p, tb):
            b = tb - tbase
            pltpu.async_copy(alpha_h.at[tb], ab[p], sin[p])
            pltpu.async_copy(xp_h.at[isa.at[b]], xr[p], sin[p])

        def wait_in(p):
            _wait(alpha_h.at[0], ab[p], sin[p])
            _wait(xp_h.at[isa.at[0]], xr[p], sin[p])

        def compute(p, tb):
            for h in range(8):
                hh = jnp.full((16,), h, jnp.int32)
                for g in range(8):
                    ii = g * 16 + iota
                    a = ab[p][h, pl.ds(g * 16, 16)]
                    e = jnp.exp(a - gvecs[h])
                    plsc.store_scatter(ex[p], [ii, hh], e)
                    for c in range(8):
                        cc = jnp.full((16,), 8 * h + c, jnp.int32)
                        xv = plsc.load_gather(xr[p], [ii, cc])
                        plsc.store_scatter(mg[p], [ii, cc], xv * e)

        def issue_out(p, tb):
            b = tb - tbase
            pltpu.async_copy(ex[p], shared_d.at[ida.at[b]], so[p], add=True)
            pltpu.async_copy(mg[p], shared_o.at[ida.at[b]], so[p], add=True)

        def wait_out(p):
            _wait(ex[p], shared_d.at[ida.at[0]], so[p])
            _wait(mg[p], shared_o.at[ida.at[0]], so[p])

        _pipeline(nblk_tile, tbase, issue_in, wait_in, compute, issue_out,
                  wait_out)
        plsc.subcore_barrier()

        @pl.when(sid == 0)
        def _():
            pltpu.sync_copy(shared_d, dparts_h.at[cid])
            pltpu.sync_copy(shared_o, oparts_h.at[cid])

    return pl.kernel(
        body,
        out_type=[_f32((NC, np_rows, 8)), _f32((NC, np_rows, 64))],
        mesh=_MESH(),
        compiler_params=_SC_PARAMS,
        scratch_types=[
            pltpu.VMEM((nblk_tile, BLK), jnp.int32),
            pltpu.VMEM((nblk_tile, BLK), jnp.int32),
            pltpu.VMEM((NW, 8, 16), jnp.float32),
            pltpu.VMEM((8, BLK), jnp.float32),
            pltpu.VMEM((8, BLK), jnp.float32),
            pltpu.VMEM((BLK, 64), jnp.float32),
            pltpu.VMEM((BLK, 64), jnp.float32),
            pltpu.VMEM((BLK, 8), jnp.float32),
            pltpu.VMEM((BLK, 8), jnp.float32),
            pltpu.VMEM((BLK, 64), jnp.float32),
            pltpu.VMEM((BLK, 64), jnp.float32),
            pltpu.VMEM_SHARED((np_rows, 8), jnp.float32),
            pltpu.VMEM_SHARED((np_rows, 64), jnp.float32),
            pltpu.SemaphoreType.DMA,
            pltpu.SemaphoreType.DMA,
            pltpu.SemaphoreType.DMA,
            pltpu.SemaphoreType.DMA,
        ],
    )


def _sc_a2(nblk_tile, np_rows):
    def body(src_h, dst_h, as_h, ad_h, alpha_h, tmax_h,
             as_v, ad_v, isa, ida, ab0, ab1, mbuf, so0, so1):
        wid = _worker_id()
        tbase = wid * nblk_tile
        pltpu.sync_copy(as_h, as_v)
        pltpu.sync_copy(ad_h, ad_v)
        pltpu.sync_copy(src_h.at[pl.ds(tbase, nblk_tile)], isa)
        pltpu.sync_copy(dst_h.at[pl.ds(tbase, nblk_tile)], ida)
        mbuf[:] = jnp.full((16,), NEG_INIT, jnp.float32)
        ab = (ab0, ab1)
        so = (so0, so1)

        def issue_in(p, tb):
            pass

        def wait_in(p):
            pass

        def compute(p, tb):
            b = tb - tbase
            mv = mbuf[:]
            for g in range(8):
                si = isa[b, pl.ds(g * 16, 16)]
                di = ida[b, pl.ds(g * 16, 16)]
                v = plsc.load_gather(as_v, [si]) + plsc.load_gather(ad_v, [di])
                a = jnp.maximum(v, 0.2 * v)
                mv = jnp.maximum(mv, a)
                ab[p][pl.ds(g * 16, 16)] = a
            mbuf[:] = mv

        def issue_out(p, tb):
            pltpu.async_copy(ab[p], alpha_h.at[tb], so[p])

        def wait_out(p):
            _wait(ab[p], alpha_h.at[0], so[p])

        _pipeline(nblk_tile, tbase, issue_in, wait_in, compute, issue_out,
                  wait_out)
        pltpu.sync_copy(mbuf, tmax_h.at[wid])

    nblk = nblk_tile * NW
    return pl.kernel(
        body,
        out_type=[_f32((nblk, BLK)), _f32((NW, 16))],
        mesh=_MESH(),
        compiler_params=_SC_PARAMS,
        scratch_types=[
            pltpu.VMEM((np_rows,), jnp.float32),
            pltpu.VMEM((np_rows,), jnp.float32),
            pltpu.VMEM((nblk_tile, BLK), jnp.int32),
            pltpu.VMEM((nblk_tile, BLK), jnp.int32),
            pltpu.VMEM((BLK,), jnp.float32),
            pltpu.VMEM((BLK,), jnp.float32),
            pltpu.VMEM((16,), jnp.float32),
            pltpu.SemaphoreType.DMA,
            pltpu.SemaphoreType.DMA,
        ],
    )


def _sc_bc2(nblk_tile, np_rows):
    def body(src_h, dst_h, alpha_h, tmax_h, xp_h, zeros8_h,
             dparts_h, oparts_h,
             isa, ida, tmax_v, ab0, ab1, xr0, xr1, ex0, ex1,
             mg0, mg1, shared_d, shared_o, sin0, sin1, so0, so1):
        cid = lax.axis_index("c")
        sid = lax.axis_index("s")
        wid = _worker_id()
        iota = _iota16()
        tbase = wid * nblk_tile
        pltpu.sync_copy(src_h.at[pl.ds(tbase, nblk_tile)], isa)
        pltpu.sync_copy(dst_h.at[pl.ds(tbase, nblk_tile)], ida)
        pltpu.sync_copy(tmax_h, tmax_v)
        gvec = _gmax_vec1(tmax_v)
        pltpu.sync_copy(zeros8_h.at[pl.ds(0, BLK)], ex0)
        pltpu.sync_copy(zeros8_h.at[pl.ds(0, BLK)], ex1)
        ab = (ab0, ab1)
        xr = (xr0, xr1)
        ex = (ex0, ex1)
        mg = (mg0, mg1)
        sin = (sin0, sin1)
        so = (so0, so1)

        @pl.when(sid == 0)
        def _():
            pltpu.sync_copy(zeros8_h, shared_d)
            pltpu.sync_copy(zeros8_h, shared_o)
        plsc.subcore_barrier()

        zz = jnp.full((16,), 0, jnp.int32)

        def issue_in(p, tb):
            b = tb - tbase
            pltpu.async_copy(alpha_h.at[tb], ab[p], sin[p])
            pltpu.async_copy(xp_h.at[isa.at[b]], xr[p], sin[p])

        def wait_in(p):
            _wait(alpha_h.at[0], ab[p], sin[p])
            _wait(xp_h.at[isa.at[0]], xr[p], sin[p])

        def compute(p, tb):
            for g in range(8):
                ii = g * 16 + iota
                a = ab[p][pl.ds(g * 16, 16)]
                e = jnp.exp(a - gvec)
                plsc.store_scatter(ex[p], [ii, zz], e)
                for c in range(8):
                    cc = jnp.full((16,), c, jnp.int32)
                    xv = plsc.load_gather(xr[p], [ii, cc])
                    plsc.store_scatter(mg[p], [ii, cc], xv * e)

        def issue_out(p, tb):
            b = tb - tbase
            pltpu.async_copy(ex[p], shared_d.at[ida.at[b]], so[p], add=True)
            pltpu.async_copy(mg[p], shared_o.at[ida.at[b]], so[p], add=True)

        def wait_out(p):
            _wait(ex[p], shared_d.at[ida.at[0]], so[p])
            _wait(mg[p], shared_o.at[ida.at[0]], so[p])

        _pipeline(nblk_tile, tbase, issue_in, wait_in, compute, issue_out,
                  wait_out)
        plsc.subcore_barrier()

        @pl.when(sid == 0)
        def _():
            pltpu.sync_copy(shared_d, dparts_h.at[cid])
            pltpu.sync_copy(shared_o, oparts_h.at[cid])

    return pl.kernel(
        body,
        out_type=[_f32((NC, np_rows, 8)), _f32((NC, np_rows, 8))],
        mesh=_MESH(),
        compiler_params=_SC_PARAMS,
        scratch_types=[
            pltpu.VMEM((nblk_tile, BLK), jnp.int32),
            pltpu.VMEM((nblk_tile, BLK), jnp.int32),
            pltpu.VMEM((NW, 16), jnp.float32),
            pltpu.VMEM((BLK,), jnp.float32),
            pltpu.VMEM((BLK,), jnp.float32),
            pltpu.VMEM((BLK, 8), jnp.float32),
            pltpu.VMEM((BLK, 8), jnp.float32),
            pltpu.VMEM((BLK, 8), jnp.float32),
            pltpu.VMEM((BLK, 8), jnp.float32),
            pltpu.VMEM((BLK, 8), jnp.float32),
            pltpu.VMEM((BLK, 8), jnp.float32),
            pltpu.VMEM_SHARED((np_rows, 8), jnp.float32),
            pltpu.VMEM_SHARED((np_rows, 8), jnp.float32),
            pltpu.SemaphoreType.DMA,
            pltpu.SemaphoreType.DMA,
            pltpu.SemaphoreType.DMA,
            pltpu.SemaphoreType.DMA,
        ],
    )


def _tc1_body(x_ref, w_ref, asrc_ref, adst_ref, xp_ref, asad_ref):
    xp = jnp.dot(x_ref[:], w_ref[:], preferred_element_type=jnp.float32)
    xr = xp.reshape(xp.shape[0], 8, 8)
    a_s = (xr * asrc_ref[:][None]).sum(-1)
    a_d = (xr * adst_ref[:][None]).sum(-1)
    xp_ref[:] = xp
    asad_ref[:] = jnp.concatenate([a_s, a_d], axis=1)


def _tc2_body(op_ref, dp_ref, b1_ref, w2_ref, a2s_ref,
              a2d_ref, xp2_ref, asad2_ref):
    num = op_ref[0] + op_ref[1]
    den = dp_ref[0] + dp_ref[1] + 1e-16
    rows = num.shape[0]
    nrm = (num.reshape(rows, 8, 8) / den[:, :, None]).reshape(rows, 64)
    v = nrm + b1_ref[:][None]
    h = jnp.where(v > 0, v, jnp.exp(v) - 1.0)
    xp2 = jnp.dot(h, w2_ref[:], preferred_element_type=jnp.float32)
    a_s = (xp2 * a2s_ref[:][None]).sum(-1, keepdims=True)
    a_d = (xp2 * a2d_ref[:][None]).sum(-1, keepdims=True)
    zero = jnp.zeros((xp2.shape[0], 6), jnp.float32)
    xp2_ref[:] = xp2
    asad2_ref[:] = jnp.concatenate([a_s, a_d, zero], axis=1)


def _tc3_body(op_ref, dp_ref, b2_ref, out_ref):
    num = op_ref[0] + op_ref[1]
    den = dp_ref[0][:, :1] + dp_ref[1][:, :1] + 1e-16
    x7 = num[:, :7] / den + b2_ref[:][None]
    m = jnp.max(x7, axis=1, keepdims=True)
    e = jnp.exp(x7 - m)
    lse = jnp.log(jnp.sum(e, axis=1, keepdims=True))
    out_ref[:] = x7 - m - lse


def kernel(x, edge_index, W1, att_src1, att_dst1, b1,
           W2, att_src2, att_dst2, b2):
    n, f_in = x.shape
    e = edge_index.shape[1]
    ne = e + n
    ep = -(-ne // (NW * BLK)) * (NW * BLK)
    nblk = ep // BLK
    nblk_tile = nblk // NW
    np_rows = -(-(n + 1) // 16) * 16

    loops = jnp.arange(n, dtype=edge_index.dtype)
    pad = ep - ne
    srcp = jnp.concatenate(
        [edge_index[0], loops, jnp.zeros((pad,), edge_index.dtype)]
    ).reshape(nblk, BLK)
    dstp = jnp.concatenate(
        [edge_index[1], loops, jnp.full((pad,), n, edge_index.dtype)]
    ).reshape(nblk, BLK)
    zeros8 = jnp.zeros((np_rows, 8), jnp.float32)
    zeros64 = jnp.zeros((np_rows, 64), jnp.float32)
    a1s = att_src1.reshape(8, 8)
    a1d = att_dst1.reshape(8, 8)
    w2p = jnp.pad(W2, ((0, 0), (0, 1)))
    a2s = jnp.pad(att_src2.reshape(7), (0, 1))
    a2d = jnp.pad(att_dst2.reshape(7), (0, 1))

    rb = 400
    g1 = n // rb
    xp1, asad1 = pl.pallas_call(
        _tc1_body,
        grid=(g1,),
        in_specs=[
            pl.BlockSpec((rb, f_in), lambda i: (i, 0)),
            pl.BlockSpec((f_in, 64), lambda i: (0, 0)),
            pl.BlockSpec((8, 8), lambda i: (0, 0)),
            pl.BlockSpec((8, 8), lambda i: (0, 0)),
        ],
        out_specs=[
            pl.BlockSpec((rb, 64), lambda i: (i, 0)),
            pl.BlockSpec((rb, 16), lambda i: (i, 0)),
        ],
        out_shape=[_f32((n, 64)), _f32((n, 16))],
    )(x, W1, a1s, a1d)
    asad1f = jnp.concatenate(
        [asad1, jnp.zeros((np_rows - n, 16), jnp.float32)])

    alpha1, tmax1 = _sc_a1(nblk_tile, np_rows)(srcp, dstp, asad1f)
    dparts1, oparts1 = _sc_bc1(nblk_tile, np_rows)(
        srcp, dstp, alpha1, tmax1, xp1, zeros8, zeros64)

    xp2, asad2 = pl.pallas_call(
        _tc2_body,
        grid=(g1,),
        in_specs=[
            pl.BlockSpec((2, rb, 64), lambda i: (0, i, 0)),
            pl.BlockSpec((2, rb, 8), lambda i: (0, i, 0)),
            pl.BlockSpec((64,), lambda i: (0,)),
            pl.BlockSpec((64, 8), lambda i: (0, 0)),
            pl.BlockSpec((8,), lambda i: (0,)),
            pl.BlockSpec((8,), lambda i: (0,)),
        ],
        out_specs=[
            pl.BlockSpec((rb, 8), lambda i: (i, 0)),
            pl.BlockSpec((rb, 8), lambda i: (i, 0)),
        ],
        out_shape=[_f32((n, 8)), _f32((n, 8))],
    )(oparts1, dparts1, b1, w2p, a2s, a2d)
    as2f = jnp.concatenate([asad2[:, 0], jnp.zeros((np_rows - n,), jnp.float32)])
    ad2f = jnp.concatenate([asad2[:, 1], jnp.zeros((np_rows - n,), jnp.float32)])

    alpha2, tmax2 = _sc_a2(nblk_tile, np_rows)(srcp, dstp, as2f, ad2f)
    dparts2, oparts2 = _sc_bc2(nblk_tile, np_rows)(
        srcp, dstp, alpha2, tmax2, xp2, zeros8)

    (out,) = pl.pallas_call(
        _tc3_body,
        grid=(g1,),
        in_specs=[
            pl.BlockSpec((2, rb, 8), lambda i: (0, i, 0)),
            pl.BlockSpec((2, rb, 8), lambda i: (0, i, 0)),
            pl.BlockSpec((7,), lambda i: (0,)),
        ],
        out_specs=[pl.BlockSpec((rb, 7), lambda i: (i, 0))],
        out_shape=[_f32((n, 7))],
    )(oparts2, dparts2, b2)
    return out
